# baseline (speedup 1.0000x reference)
import functools

import jax
import jax.numpy as jnp
from jax import lax
from jax.experimental import pallas as pl
from jax.experimental.pallas import tpu as pltpu
from jax.experimental.pallas import tpu_sc as plsc

B = 16384
F = 26
V = 100000
D = 16
H1 = 32
H2 = 32
EPS = 1e-5

N = B * F
NC = 2
NS = 16
NW = NC * NS
PER_W = N // NW
DMA_ROWS = 128
DMAS_PER_W = PER_W // DMA_ROWS
CHUNK_DMAS = 8
NCHUNK = DMAS_PER_W // CHUNK_DMAS
CHUNK_ROWS = CHUNK_DMAS * DMA_ROWS

BT = 512
NT = B // BT


SAMP_W = B // NW
SCHUNK = 64
NCHUNK2 = SAMP_W // SCHUNK


def _sc_gather_body(idxT_hbm, w2_hbm, w1_hbm, e2_hbm, e1_hbm,
                    idxc, gbuf, e1b, semg, sem1):
    wid = lax.axis_index("s") * NC + lax.axis_index("c")

    def chunk(c, carry):
        b0 = pl.multiple_of(wid * SAMP_W + c * SCHUNK, SCHUNK)
        pltpu.sync_copy(idxT_hbm.at[:, pl.ds(b0, SCHUNK)], idxc)
        cops = []
        for f in range(F):
            cops.append(pltpu.async_copy(
                w2_hbm.at[f].at[idxc.at[f]],
                gbuf.at[pl.ds(f * SCHUNK, SCHUNK)], semg))
            cops.append(pltpu.async_copy(
                w1_hbm.at[f].at[idxc.at[f]], e1b.at[f], sem1))
        for cop in cops:
            cop.wait()
        for f in range(F):
            pltpu.sync_copy(gbuf.at[pl.ds(f * SCHUNK, SCHUNK)],
                            e2_hbm.at[pl.ds(b0, SCHUNK), pl.ds(f * D, D)])
        pltpu.sync_copy(e1b, e1_hbm.at[:, pl.ds(b0, SCHUNK)])
        return carry

    lax.fori_loop(0, NCHUNK2, chunk, 0)


@functools.cache
def _sc_gather():
    return pl.kernel(
        _sc_gather_body,
        out_type=[jax.ShapeDtypeStruct((B, F * D), jnp.float32),
                  jax.ShapeDtypeStruct((F, B), jnp.float32)],
        mesh=plsc.VectorSubcoreMesh(core_axis_name="c", subcore_axis_name="s",
                                    num_cores=NC, num_subcores=NS),
        scratch_types=[
            pltpu.VMEM((F, SCHUNK), jnp.int32),
            pltpu.VMEM((F * SCHUNK, D), jnp.float32),
            pltpu.VMEM((F, SCHUNK), jnp.float32),
            pltpu.SemaphoreType.DMA,
            pltpu.SemaphoreType.DMA,
        ],
        compiler_params=pltpu.CompilerParams(use_tc_tiling_on_sc=False),
    )


def _fm_body(e2_ref, xv_ref, e1_ref, rmat_ref, swmat_ref,
             part0_ref, hc_ref, s1_ref, cc_ref):
    i = pl.program_id(0)
    e2 = e2_ref[...]
    xv = xv_ref[...]
    e1 = e1_ref[...]
    xe = lax.dot_general(xv, rmat_ref[...], (((1,), (0,)), ((), ())),
                         preferred_element_type=jnp.float32,
                         precision=lax.Precision.HIGHEST)
    deep = e2 * xe
    m = lax.dot_general(deep, swmat_ref[...], (((1,), (0,)), ((), ())),
                        preferred_element_type=jnp.float32,
                        precision=lax.Precision.HIGHEST)
    s = m[:, :D]
    hc = m[:, D:]
    fm2 = 0.5 * (jnp.sum(s * s, axis=1) - jnp.sum(deep * deep, axis=1))
    fm1 = jnp.sum(e1 * xv, axis=1)
    part0_ref[...] = fm1 + fm2
    hc_ref[...] = hc

    @pl.when(i == 0)
    def _init():
        s1_ref[...] = jnp.zeros_like(s1_ref)
        cc_ref[...] = jnp.zeros_like(cc_ref)

    s1_ref[...] += jnp.sum(hc, axis=0, keepdims=True)
    cc_ref[...] += lax.dot_general(hc, hc, (((0,), (0,)), ((), ())),
                                   preferred_element_type=jnp.float32,
                                   precision=lax.Precision.HIGHEST)


def _fm_call(e2m, xv, e1m, rmat, swmat):
    return pl.pallas_call(
        _fm_body,
        grid=(NT,),
        in_specs=[
            pl.BlockSpec((BT, F * D), lambda i: (i, 0)),
            pl.BlockSpec((BT, F), lambda i: (i, 0)),
            pl.BlockSpec((BT, F), lambda i: (i, 0)),
            pl.BlockSpec((F, F * D), lambda i: (0, 0)),
            pl.BlockSpec((F * D, D + H1), lambda i: (0, 0)),
        ],
        out_specs=[
            pl.BlockSpec((BT,), lambda i: (i,)),
            pl.BlockSpec((BT, H1), lambda i: (i, 0)),
            pl.BlockSpec((1, H1), lambda i: (0, 0)),
            pl.BlockSpec((H1, H1), lambda i: (0, 0)),
        ],
        out_shape=[
            jax.ShapeDtypeStruct((B,), jnp.float32),
            jax.ShapeDtypeStruct((B, H1), jnp.float32),
            jax.ShapeDtypeStruct((1, H1), jnp.float32),
            jax.ShapeDtypeStruct((H1, H1), jnp.float32),
        ],
    )(e2m, xv, e1m, rmat, swmat)


def _fin_body(part0_ref, hc_ref, s1_ref, cc_ref, bias_ref, bl1_ref, g1_ref,
              b1_ref, wl2_ref, bl2_ref, g2_ref, b2_ref, out_ref):
    mc = s1_ref[...] * (1.0 / B)
    cc = cc_ref[...]
    eye = (lax.broadcasted_iota(jnp.int32, (H1, H1), 0)
           == lax.broadcasted_iota(jnp.int32, (H1, H1), 1)).astype(jnp.float32)
    diag = jnp.sum(cc * eye, axis=0, keepdims=True)
    v1 = diag * (1.0 / B) - mc * mc
    m1 = mc + bl1_ref[...]
    a = g1_ref[...] * lax.rsqrt(v1 + EPS)
    wl2 = wl2_ref[...]

    def rowvec_matT(x):
        return lax.dot_general(x, wl2, (((1,), (1,)), ((), ())),
                               preferred_element_type=jnp.float32,
                         precision=lax.Precision.HIGHEST)

    c = rowvec_matT(b1_ref[...] - m1 * a) + bl2_ref[...]
    m2 = rowvec_matT(m1 * a) + c
    outer_mc = lax.dot_general(mc, mc, (((0,), (0,)), ((), ())),
                               preferred_element_type=jnp.float32,
                         precision=lax.Precision.HIGHEST)
    cov = cc * (1.0 / B) - outer_mc
    outer_a = lax.dot_general(a, a, (((0,), (0,)), ((), ())),
                              preferred_element_type=jnp.float32,
                         precision=lax.Precision.HIGHEST)
    p = cov * outer_a
    q = lax.dot_general(wl2, p, (((1,), (0,)), ((), ())),
                        preferred_element_type=jnp.float32,
                         precision=lax.Precision.HIGHEST)
    v2 = jnp.sum(q * wl2, axis=1).reshape(1, H2)
    w2v = g2_ref[...] * lax.rsqrt(v2 + EPS)
    u = a * lax.dot_general(w2v, wl2, (((1,), (0,)), ((), ())),
                            preferred_element_type=jnp.float32,
                         precision=lax.Precision.HIGHEST)
    const = (bias_ref[0, 0]
             + jnp.sum((c - m2) * w2v)
             + jnp.sum(b2_ref[...])
             + jnp.sum(bl1_ref[...] * u))
    mv = lax.dot_general(hc_ref[...], u, (((1,), (1,)), ((), ())),
                         preferred_element_type=jnp.float32,
                         precision=lax.Precision.HIGHEST)
    out_ref[...] = part0_ref[...] + jnp.sum(mv, axis=1) + const


def _fin_call(part0, hc, s1, cc, bias, bl1, g1, b1, wl2, bl2, g2, b2):
    return pl.pallas_call(
        _fin_body,
        out_shape=jax.ShapeDtypeStruct((B,), jnp.float32),
    )(part0, hc, s1, cc, bias, bl1, g1, b1, wl2, bl2, g2, b2)


def kernel(Xi, Xv, W1, W2, bias, Wl1, bl1, g1, b1, Wl2, bl2, g2, b2):
    idxT = jnp.transpose(Xi[:, :, 0])
    e2m, e1T = _sc_gather()(idxT, W2, W1.reshape(F, V))
    e1m = jnp.transpose(e1T)

    col = jnp.arange(F * D, dtype=jnp.int32)
    rmat = (col[None, :] // D == jnp.arange(F, dtype=jnp.int32)[:, None]
            ).astype(jnp.float32)
    smat = (col[:, None] % D == jnp.arange(D, dtype=jnp.int32)[None, :]
            ).astype(jnp.float32)
    swmat = jnp.concatenate([smat, Wl1.T], axis=1)

    part0, hc, s1, cc = _fm_call(e2m, Xv, e1m, rmat, swmat)
    out = _fin_call(part0, hc, s1, cc,
                    bias.reshape(1, 1), bl1.reshape(1, H1),
                    g1.reshape(1, H1), b1.reshape(1, H1), Wl2,
                    bl2.reshape(1, H2), g2.reshape(1, H2), b2.reshape(1, H2))
    return out


# speedup vs baseline: 2.5185x; 2.5185x over previous
import functools

import jax
import jax.numpy as jnp
from jax import lax
from jax.experimental import pallas as pl
from jax.experimental.pallas import tpu as pltpu
from jax.experimental.pallas import tpu_sc as plsc

B = 16384
F = 26
V = 100000
D = 16
H1 = 32
H2 = 32
EPS = 1e-5

N = B * F
NC = 2
NS = 16
NW = NC * NS
PER_W = N // NW
DMA_ROWS = 128
DMAS_PER_W = PER_W // DMA_ROWS
CHUNK_DMAS = 8
NCHUNK = DMAS_PER_W // CHUNK_DMAS
CHUNK_ROWS = CHUNK_DMAS * DMA_ROWS

BT = 512
NT = B // BT


VC = 12800
NVC = 8
NFG = 4
ROWS16 = NFG * NVC * VC * 8


def _detile_body(w2t_ref, out_ref):
    out_ref[...] = jnp.transpose(w2t_ref[...])


def _detile_call(w2t2):
    return pl.pallas_call(
        _detile_body,
        grid=(NFG, NVC),
        in_specs=[pl.BlockSpec((128, VC), lambda g, c: (g, c))],
        out_specs=pl.BlockSpec((VC, 128), lambda g, c: (g * NVC + c, 0)),
        out_shape=jax.ShapeDtypeStruct((NFG * NVC * VC, 128), jnp.float32),
    )(w2t2)


SAMP_W = B // NW
SCHUNK = 64
NCHUNK2 = SAMP_W // SCHUNK


def _sc_gather_body(idxP_hbm, idxT_hbm, w2_hbm, w1_hbm, e2_hbm, e1_hbm,
                    idxc, idxr, gbuf, e1b, semg, sem1):
    wid = lax.axis_index("s") * NC + lax.axis_index("c")

    def chunk(c, carry):
        b0 = pl.multiple_of(wid * SAMP_W + c * SCHUNK, SCHUNK)
        pltpu.sync_copy(idxP_hbm.at[:, pl.ds(b0, SCHUNK)], idxc)
        pltpu.sync_copy(idxT_hbm.at[:, pl.ds(b0, SCHUNK)], idxr)
        cops = []
        for f in range(F):
            cops.append(pltpu.async_copy(
                w2_hbm.at[pl.ds((f // 8) * NVC * VC * 8, NVC * VC * 8)]
                .at[idxc.at[f]],
                gbuf.at[pl.ds(f * SCHUNK, SCHUNK)], semg))
            cops.append(pltpu.async_copy(
                w1_hbm.at[f].at[idxr.at[f]], e1b.at[f], sem1))
        for cop in cops:
            cop.wait()
        for f in range(F):
            pltpu.sync_copy(gbuf.at[pl.ds(f * SCHUNK, SCHUNK)],
                            e2_hbm.at[pl.ds(b0, SCHUNK), pl.ds(f * D, D)])
        pltpu.sync_copy(e1b, e1_hbm.at[:, pl.ds(b0, SCHUNK)])
        return carry

    lax.fori_loop(0, NCHUNK2, chunk, 0)


@functools.cache
def _sc_gather():
    return pl.kernel(
        _sc_gather_body,
        out_type=[jax.ShapeDtypeStruct((B, F * D), jnp.float32),
                  jax.ShapeDtypeStruct((F, B), jnp.float32)],
        mesh=plsc.VectorSubcoreMesh(core_axis_name="c", subcore_axis_name="s",
                                    num_cores=NC, num_subcores=NS),
        scratch_types=[
            pltpu.VMEM((F, SCHUNK), jnp.int32),
            pltpu.VMEM((F, SCHUNK), jnp.int32),
            pltpu.VMEM((F * SCHUNK, D), jnp.float32),
            pltpu.VMEM((F, SCHUNK), jnp.float32),
            pltpu.SemaphoreType.DMA,
            pltpu.SemaphoreType.DMA,
        ],
        compiler_params=pltpu.CompilerParams(use_tc_tiling_on_sc=False),
    )


def _fm_body(e2_ref, xvt_ref, e1t_ref, rmat_ref, swmat_ref,
             part0_ref, hc_ref, s1_ref, cc_ref):
    i = pl.program_id(0)
    e2 = e2_ref[...]
    xvt = xvt_ref[...]
    e1t = e1t_ref[...]
    xe = lax.dot_general(xvt, rmat_ref[...], (((0,), (0,)), ((), ())),
                         preferred_element_type=jnp.float32,
                         precision=lax.Precision.HIGHEST)
    deep = e2 * xe
    m = lax.dot_general(deep, swmat_ref[...], (((1,), (0,)), ((), ())),
                        preferred_element_type=jnp.float32,
                        precision=lax.Precision.HIGHEST)
    s = m[:, :D]
    hc = m[:, D:]
    fm2 = 0.5 * (jnp.sum(s * s, axis=1) - jnp.sum(deep * deep, axis=1))
    fm1 = jnp.sum(e1t * xvt, axis=0)
    part0_ref[...] = fm1 + fm2
    hc_ref[...] = hc

    @pl.when(i == 0)
    def _init():
        s1_ref[...] = jnp.zeros_like(s1_ref)
        cc_ref[...] = jnp.zeros_like(cc_ref)

    s1_ref[...] += jnp.sum(hc, axis=0, keepdims=True)
    cc_ref[...] += lax.dot_general(hc, hc, (((0,), (0,)), ((), ())),
                                   preferred_element_type=jnp.float32,
                                   precision=lax.Precision.HIGHEST)


def _fm_call(e2m, xvt, e1t, rmat, swmat):
    return pl.pallas_call(
        _fm_body,
        grid=(NT,),
        in_specs=[
            pl.BlockSpec((BT, F * D), lambda i: (i, 0)),
            pl.BlockSpec((F, BT), lambda i: (0, i)),
            pl.BlockSpec((F, BT), lambda i: (0, i)),
            pl.BlockSpec((F, F * D), lambda i: (0, 0)),
            pl.BlockSpec((F * D, D + H1), lambda i: (0, 0)),
        ],
        out_specs=[
            pl.BlockSpec((BT,), lambda i: (i,)),
            pl.BlockSpec((BT, H1), lambda i: (i, 0)),
            pl.BlockSpec((1, H1), lambda i: (0, 0)),
            pl.BlockSpec((H1, H1), lambda i: (0, 0)),
        ],
        out_shape=[
            jax.ShapeDtypeStruct((B,), jnp.float32),
            jax.ShapeDtypeStruct((B, H1), jnp.float32),
            jax.ShapeDtypeStruct((1, H1), jnp.float32),
            jax.ShapeDtypeStruct((H1, H1), jnp.float32),
        ],
    )(e2m, xvt, e1t, rmat, swmat)


def _fin_body(part0_ref, hc_ref, s1_ref, cc_ref, bias_ref, bl1_ref, g1_ref,
              b1_ref, wl2_ref, bl2_ref, g2_ref, b2_ref, out_ref):
    mc = s1_ref[...] * (1.0 / B)
    cc = cc_ref[...]
    eye = (lax.broadcasted_iota(jnp.int32, (H1, H1), 0)
           == lax.broadcasted_iota(jnp.int32, (H1, H1), 1)).astype(jnp.float32)
    diag = jnp.sum(cc * eye, axis=0, keepdims=True)
    v1 = diag * (1.0 / B) - mc * mc
    m1 = mc + bl1_ref[...]
    a = g1_ref[...] * lax.rsqrt(v1 + EPS)
    wl2 = wl2_ref[...]

    def rowvec_matT(x):
        return lax.dot_general(x, wl2, (((1,), (1,)), ((), ())),
                               preferred_element_type=jnp.float32,
                         precision=lax.Precision.HIGHEST)

    c = rowvec_matT(b1_ref[...] - m1 * a) + bl2_ref[...]
    m2 = rowvec_matT(m1 * a) + c
    outer_mc = lax.dot_general(mc, mc, (((0,), (0,)), ((), ())),
                               preferred_element_type=jnp.float32,
                         precision=lax.Precision.HIGHEST)
    cov = cc * (1.0 / B) - outer_mc
    outer_a = lax.dot_general(a, a, (((0,), (0,)), ((), ())),
                              preferred_element_type=jnp.float32,
                         precision=lax.Precision.HIGHEST)
    p = cov * outer_a
    q = lax.dot_general(wl2, p, (((1,), (0,)), ((), ())),
                        preferred_element_type=jnp.float32,
                         precision=lax.Precision.HIGHEST)
    v2 = jnp.sum(q * wl2, axis=1).reshape(1, H2)
    w2v = g2_ref[...] * lax.rsqrt(v2 + EPS)
    u = a * lax.dot_general(w2v, wl2, (((1,), (0,)), ((), ())),
                            preferred_element_type=jnp.float32,
                         precision=lax.Precision.HIGHEST)
    const = (bias_ref[0, 0]
             + jnp.sum((c - m2) * w2v)
             + jnp.sum(b2_ref[...])
             + jnp.sum(bl1_ref[...] * u))
    mv = lax.dot_general(hc_ref[...], u, (((1,), (1,)), ((), ())),
                         preferred_element_type=jnp.float32,
                         precision=lax.Precision.HIGHEST)
    out_ref[...] = part0_ref[...] + jnp.sum(mv, axis=1) + const


def _fin_call(part0, hc, s1, cc, bias, bl1, g1, b1, wl2, bl2, g2, b2):
    return pl.pallas_call(
        _fin_body,
        out_shape=jax.ShapeDtypeStruct((B,), jnp.float32),
    )(part0, hc, s1, cc, bias, bl1, g1, b1, wl2, bl2, g2, b2)


def kernel(Xi, Xv, W1, W2, bias, Wl1, bl1, g1, b1, Wl2, bl2, g2, b2):
    idxT = jnp.transpose(Xi[:, :, 0])
    fcol = jnp.arange(F, dtype=jnp.int32)[:, None]
    idxP = ((idxT // VC) * (VC * 8) + (idxT % VC) * 8 + fcol % 8)
    w2t2 = jnp.transpose(W2, (0, 2, 1)).reshape(F * D, V)
    w2l = _detile_call(w2t2).reshape(ROWS16, D)
    e2m, e1T = _sc_gather()(idxP, idxT, w2l, W1.reshape(F, V))
    xvT = jnp.transpose(Xv)

    col = jnp.arange(F * D, dtype=jnp.int32)
    rmat = (col[None, :] // D == jnp.arange(F, dtype=jnp.int32)[:, None]
            ).astype(jnp.float32)
    smat = (col[:, None] % D == jnp.arange(D, dtype=jnp.int32)[None, :]
            ).astype(jnp.float32)
    swmat = jnp.concatenate([smat, Wl1.T], axis=1)

    part0, hc, s1, cc = _fm_call(e2m, xvT, e1T, rmat, swmat)
    out = _fin_call(part0, hc, s1, cc,
                    bias.reshape(1, 1), bl1.reshape(1, H1),
                    g1.reshape(1, H1), b1.reshape(1, H1), Wl2,
                    bl2.reshape(1, H2), g2.reshape(1, H2), b2.reshape(1, H2))
    return out


# speedup vs baseline: 2.5886x; 1.0278x over previous
import functools

import jax
import jax.numpy as jnp
from jax import lax
from jax.experimental import pallas as pl
from jax.experimental.pallas import tpu as pltpu
from jax.experimental.pallas import tpu_sc as plsc

B = 16384
F = 26
V = 100000
D = 16
H1 = 32
H2 = 32
EPS = 1e-5

N = B * F
NC = 2
NS = 16
NW = NC * NS
PER_W = N // NW
DMA_ROWS = 128
DMAS_PER_W = PER_W // DMA_ROWS
CHUNK_DMAS = 8
NCHUNK = DMAS_PER_W // CHUNK_DMAS
CHUNK_ROWS = CHUNK_DMAS * DMA_ROWS

BT = 512
NT = B // BT


VC = 12800
NVC = 8
NFG = 4
ROWS16 = NFG * NVC * VC * 8


def _detile_body(w2t_ref, out_ref):
    out_ref[...] = jnp.transpose(w2t_ref[...])


def _detile_call(w2t2):
    return pl.pallas_call(
        _detile_body,
        grid=(NFG, NVC),
        in_specs=[pl.BlockSpec((128, VC), lambda g, c: (g, c))],
        out_specs=pl.BlockSpec((VC, 128), lambda g, c: (g * NVC + c, 0)),
        out_shape=jax.ShapeDtypeStruct((NFG * NVC * VC, 128), jnp.float32),
    )(w2t2)


SAMP_W = B // NW
SCHUNK = 64
NCHUNK2 = SAMP_W // SCHUNK


def _sc_gather_body(idxP_hbm, idxT_hbm, w2_hbm, w1_hbm, e2_hbm, e1_hbm,
                    idxc, idxr, gbuf, e1b, semg, sem1):
    wid = lax.axis_index("s") * NC + lax.axis_index("c")

    def chunk(c, carry):
        b0 = pl.multiple_of(wid * SAMP_W + c * SCHUNK, SCHUNK)
        pltpu.sync_copy(idxP_hbm.at[:, pl.ds(b0, SCHUNK)], idxc)
        pltpu.sync_copy(idxT_hbm.at[:, pl.ds(b0, SCHUNK)], idxr)
        cops = []
        for f in range(F):
            cops.append(pltpu.async_copy(
                w2_hbm.at[pl.ds((f // 8) * NVC * VC * 8, NVC * VC * 8)]
                .at[idxc.at[f]],
                gbuf.at[pl.ds(f * SCHUNK, SCHUNK)], semg))
            cops.append(pltpu.async_copy(
                w1_hbm.at[f, 0].at[idxr.at[f]], e1b.at[f], sem1))
        for cop in cops:
            cop.wait()
        for f in range(F):
            pltpu.sync_copy(gbuf.at[pl.ds(f * SCHUNK, SCHUNK)],
                            e2_hbm.at[pl.ds(b0, SCHUNK), pl.ds(f * D, D)])
        pltpu.sync_copy(e1b, e1_hbm.at[:, pl.ds(b0, SCHUNK)])
        return carry

    lax.fori_loop(0, NCHUNK2, chunk, 0)


@functools.cache
def _sc_gather():
    return pl.kernel(
        _sc_gather_body,
        out_type=[jax.ShapeDtypeStruct((B, F * D), jnp.float32),
                  jax.ShapeDtypeStruct((F, B), jnp.float32)],
        mesh=plsc.VectorSubcoreMesh(core_axis_name="c", subcore_axis_name="s",
                                    num_cores=NC, num_subcores=NS),
        scratch_types=[
            pltpu.VMEM((F, SCHUNK), jnp.int32),
            pltpu.VMEM((F, SCHUNK), jnp.int32),
            pltpu.VMEM((F * SCHUNK, D), jnp.float32),
            pltpu.VMEM((F, SCHUNK), jnp.float32),
            pltpu.SemaphoreType.DMA,
            pltpu.SemaphoreType.DMA,
        ],
        compiler_params=pltpu.CompilerParams(use_tc_tiling_on_sc=False),
    )


def _fm_body(e2_ref, xvt_ref, e1t_ref, rmat_ref, swmat_ref,
             part0_ref, hc_ref, s1_ref, cc_ref):
    i = pl.program_id(0)
    e2 = e2_ref[...]
    xvt = xvt_ref[...]
    e1t = e1t_ref[...]
    xe = lax.dot_general(xvt, rmat_ref[...], (((0,), (0,)), ((), ())),
                         preferred_element_type=jnp.float32,
                         precision=lax.Precision.HIGHEST)
    deep = e2 * xe
    m = lax.dot_general(deep, swmat_ref[...], (((1,), (0,)), ((), ())),
                        preferred_element_type=jnp.float32,
                        precision=lax.Precision.HIGHEST)
    s = m[:, :D]
    hc = m[:, D:]
    fm2 = 0.5 * (jnp.sum(s * s, axis=1) - jnp.sum(deep * deep, axis=1))
    fm1 = jnp.sum(e1t * xvt, axis=0)
    part0_ref[...] = fm1 + fm2
    hc_ref[...] = hc

    @pl.when(i == 0)
    def _init():
        s1_ref[...] = jnp.zeros_like(s1_ref)
        cc_ref[...] = jnp.zeros_like(cc_ref)

    s1_ref[...] += jnp.sum(hc, axis=0, keepdims=True)
    cc_ref[...] += lax.dot_general(hc, hc, (((0,), (0,)), ((), ())),
                                   preferred_element_type=jnp.float32,
                                   precision=lax.Precision.HIGHEST)


def _fm_call(e2m, xvt, e1t, rmat, swmat):
    return pl.pallas_call(
        _fm_body,
        grid=(NT,),
        in_specs=[
            pl.BlockSpec((BT, F * D), lambda i: (i, 0)),
            pl.BlockSpec((F, BT), lambda i: (0, i)),
            pl.BlockSpec((F, BT), lambda i: (0, i)),
            pl.BlockSpec((F, F * D), lambda i: (0, 0)),
            pl.BlockSpec((F * D, D + H1), lambda i: (0, 0)),
        ],
        out_specs=[
            pl.BlockSpec((BT,), lambda i: (i,)),
            pl.BlockSpec((BT, H1), lambda i: (i, 0)),
            pl.BlockSpec((1, H1), lambda i: (0, 0)),
            pl.BlockSpec((H1, H1), lambda i: (0, 0)),
        ],
        out_shape=[
            jax.ShapeDtypeStruct((B,), jnp.float32),
            jax.ShapeDtypeStruct((B, H1), jnp.float32),
            jax.ShapeDtypeStruct((1, H1), jnp.float32),
            jax.ShapeDtypeStruct((H1, H1), jnp.float32),
        ],
    )(e2m, xvt, e1t, rmat, swmat)


def _fin_body(part0_ref, hc_ref, s1_ref, cc_ref, bias_ref, bl1_ref, g1_ref,
              b1_ref, wl2_ref, bl2_ref, g2_ref, b2_ref, out_ref):
    mc = s1_ref[...] * (1.0 / B)
    cc = cc_ref[...]
    eye = (lax.broadcasted_iota(jnp.int32, (H1, H1), 0)
           == lax.broadcasted_iota(jnp.int32, (H1, H1), 1)).astype(jnp.float32)
    diag = jnp.sum(cc * eye, axis=0, keepdims=True)
    v1 = diag * (1.0 / B) - mc * mc
    m1 = mc + bl1_ref[...]
    a = g1_ref[...] * lax.rsqrt(v1 + EPS)
    wl2 = wl2_ref[...]

    def rowvec_matT(x):
        return lax.dot_general(x, wl2, (((1,), (1,)), ((), ())),
                               preferred_element_type=jnp.float32,
                         precision=lax.Precision.HIGHEST)

    c = rowvec_matT(b1_ref[...] - m1 * a) + bl2_ref[...]
    m2 = rowvec_matT(m1 * a) + c
    outer_mc = lax.dot_general(mc, mc, (((0,), (0,)), ((), ())),
                               preferred_element_type=jnp.float32,
                         precision=lax.Precision.HIGHEST)
    cov = cc * (1.0 / B) - outer_mc
    outer_a = lax.dot_general(a, a, (((0,), (0,)), ((), ())),
                              preferred_element_type=jnp.float32,
                         precision=lax.Precision.HIGHEST)
    p = cov * outer_a
    q = lax.dot_general(wl2, p, (((1,), (0,)), ((), ())),
                        preferred_element_type=jnp.float32,
                         precision=lax.Precision.HIGHEST)
    v2 = jnp.sum(q * wl2, axis=1).reshape(1, H2)
    w2v = g2_ref[...] * lax.rsqrt(v2 + EPS)
    u = a * lax.dot_general(w2v, wl2, (((1,), (0,)), ((), ())),
                            preferred_element_type=jnp.float32,
                         precision=lax.Precision.HIGHEST)
    const = (bias_ref[0, 0]
             + jnp.sum((c - m2) * w2v)
             + jnp.sum(b2_ref[...])
             + jnp.sum(bl1_ref[...] * u))
    mv = lax.dot_general(hc_ref[...], u, (((1,), (1,)), ((), ())),
                         preferred_element_type=jnp.float32,
                         precision=lax.Precision.HIGHEST)
    out_ref[...] = part0_ref[...] + jnp.sum(mv, axis=1) + const


def _fin_call(part0, hc, s1, cc, bias, bl1, g1, b1, wl2, bl2, g2, b2):
    return pl.pallas_call(
        _fin_body,
        out_shape=jax.ShapeDtypeStruct((B,), jnp.float32),
    )(part0, hc, s1, cc, bias, bl1, g1, b1, wl2, bl2, g2, b2)


def kernel(Xi, Xv, W1, W2, bias, Wl1, bl1, g1, b1, Wl2, bl2, g2, b2):
    idxT = jnp.transpose(Xi[:, :, 0])
    fcol = jnp.arange(F, dtype=jnp.int32)[:, None]
    idxP = ((idxT // VC) * (VC * 8) + (idxT % VC) * 8 + fcol % 8)
    w2t2 = jnp.transpose(W2, (0, 2, 1)).reshape(F * D, V)
    w2l = _detile_call(w2t2).reshape(ROWS16, D)
    w1t = jnp.transpose(W1, (0, 2, 1))
    e2m, e1T = _sc_gather()(idxP, idxT, w2l, w1t)
    xvT = jnp.transpose(Xv)

    col = jnp.arange(F * D, dtype=jnp.int32)
    rmat = (col[None, :] // D == jnp.arange(F, dtype=jnp.int32)[:, None]
            ).astype(jnp.float32)
    smat = (col[:, None] % D == jnp.arange(D, dtype=jnp.int32)[None, :]
            ).astype(jnp.float32)
    swmat = jnp.concatenate([smat, Wl1.T], axis=1)

    part0, hc, s1, cc = _fm_call(e2m, xvT, e1T, rmat, swmat)
    out = _fin_call(part0, hc, s1, cc,
                    bias.reshape(1, 1), bl1.reshape(1, H1),
                    g1.reshape(1, H1), b1.reshape(1, H1), Wl2,
                    bl2.reshape(1, H2), g2.reshape(1, H2), b2.reshape(1, H2))
    return out


# speedup vs baseline: 2.9701x; 1.1474x over previous
import functools

import jax
import jax.numpy as jnp
from jax import lax
from jax.experimental import pallas as pl
from jax.experimental.pallas import tpu as pltpu
from jax.experimental.pallas import tpu_sc as plsc

B = 16384
F = 26
V = 100000
D = 16
H1 = 32
H2 = 32
EPS = 1e-5

N = B * F
NC = 2
NS = 16
NW = NC * NS
PER_W = N // NW
DMA_ROWS = 128
DMAS_PER_W = PER_W // DMA_ROWS
CHUNK_DMAS = 8
NCHUNK = DMAS_PER_W // CHUNK_DMAS
CHUNK_ROWS = CHUNK_DMAS * DMA_ROWS

BT = 512
NT = B // BT


VC = 12800
NVC = 8
NFG = 4
ROWS16 = NFG * NVC * VC * 8


def _detile_body(w2t_ref, out_ref):
    out_ref[...] = jnp.transpose(w2t_ref[...])


def _detile_call(w2t2):
    return pl.pallas_call(
        _detile_body,
        grid=(NFG, NVC),
        in_specs=[pl.BlockSpec((128, VC), lambda g, c: (g, c))],
        out_specs=pl.BlockSpec((VC, 128), lambda g, c: (g * NVC + c, 0)),
        out_shape=jax.ShapeDtypeStruct((NFG * NVC * VC, 128), jnp.float32),
    )(w2t2)


SAMP_W = B // NW
SCHUNK = 64
NCHUNK2 = SAMP_W // SCHUNK


def _sc_gather_e2_body(idxP_hbm, w2_hbm, e2_hbm, idxc, gbuf, semg):
    wid = lax.axis_index("s") * NC + lax.axis_index("c")

    def chunk(c, carry):
        b0 = pl.multiple_of(wid * SAMP_W + c * SCHUNK, SCHUNK)
        pltpu.sync_copy(idxP_hbm.at[:, pl.ds(b0, SCHUNK)], idxc)
        cops = []
        for f in range(F):
            cops.append(pltpu.async_copy(
                w2_hbm.at[pl.ds((f // 8) * NVC * VC * 8, NVC * VC * 8)]
                .at[idxc.at[f]],
                gbuf.at[pl.ds(f * SCHUNK, SCHUNK)], semg))
        for cop in cops:
            cop.wait()
        for f in range(F):
            pltpu.sync_copy(gbuf.at[pl.ds(f * SCHUNK, SCHUNK)],
                            e2_hbm.at[pl.ds(b0, SCHUNK), pl.ds(f * D, D)])
        return carry

    lax.fori_loop(0, NCHUNK2, chunk, 0)


def _sc_gather_e1_body(idxT_hbm, w1_hbm, e1_hbm, idxr, e1b, sem1):
    wid = lax.axis_index("s") * NC + lax.axis_index("c")

    def chunk(c, carry):
        b0 = pl.multiple_of(wid * SAMP_W + c * SCHUNK, SCHUNK)
        pltpu.sync_copy(idxT_hbm.at[:, pl.ds(b0, SCHUNK)], idxr)
        cops = []
        for f in range(F):
            cops.append(pltpu.async_copy(
                w1_hbm.at[f, 0].at[idxr.at[f]], e1b.at[f], sem1))
        for cop in cops:
            cop.wait()
        pltpu.sync_copy(e1b, e1_hbm.at[:, pl.ds(b0, SCHUNK)])
        return carry

    lax.fori_loop(0, NCHUNK2, chunk, 0)


@functools.cache
def _sc_gather_e2():
    return pl.kernel(
        _sc_gather_e2_body,
        out_type=jax.ShapeDtypeStruct((B, F * D), jnp.float32),
        mesh=plsc.VectorSubcoreMesh(core_axis_name="c", subcore_axis_name="s",
                                    num_cores=NC, num_subcores=NS),
        scratch_types=[
            pltpu.VMEM((F, SCHUNK), jnp.int32),
            pltpu.VMEM((F * SCHUNK, D), jnp.float32),
            pltpu.SemaphoreType.DMA,
        ],
        compiler_params=pltpu.CompilerParams(use_tc_tiling_on_sc=False),
    )


@functools.cache
def _sc_gather_e1():
    return pl.kernel(
        _sc_gather_e1_body,
        out_type=jax.ShapeDtypeStruct((F, B), jnp.float32),
        mesh=plsc.VectorSubcoreMesh(core_axis_name="c", subcore_axis_name="s",
                                    num_cores=NC, num_subcores=NS),
        scratch_types=[
            pltpu.VMEM((F, SCHUNK), jnp.int32),
            pltpu.VMEM((F, SCHUNK), jnp.float32),
            pltpu.SemaphoreType.DMA,
        ],
        compiler_params=pltpu.CompilerParams(use_tc_tiling_on_sc=False),
    )


def _fm_body(e2_ref, xvt_ref, e1t_ref, rmat_ref, swmat_ref,
             part0_ref, hc_ref, s1_ref, cc_ref):
    i = pl.program_id(0)
    e2 = e2_ref[...]
    xvt = xvt_ref[...]
    e1t = e1t_ref[...]
    xe = lax.dot_general(xvt, rmat_ref[...], (((0,), (0,)), ((), ())),
                         preferred_element_type=jnp.float32,
                         precision=lax.Precision.HIGHEST)
    deep = e2 * xe
    m = lax.dot_general(deep, swmat_ref[...], (((1,), (0,)), ((), ())),
                        preferred_element_type=jnp.float32,
                        precision=lax.Precision.HIGHEST)
    s = m[:, :D]
    hc = m[:, D:]
    fm2 = 0.5 * (jnp.sum(s * s, axis=1) - jnp.sum(deep * deep, axis=1))
    fm1 = jnp.sum(e1t * xvt, axis=0)
    part0_ref[...] = fm1 + fm2
    hc_ref[...] = hc

    @pl.when(i == 0)
    def _init():
        s1_ref[...] = jnp.zeros_like(s1_ref)
        cc_ref[...] = jnp.zeros_like(cc_ref)

    s1_ref[...] += jnp.sum(hc, axis=0, keepdims=True)
    cc_ref[...] += lax.dot_general(hc, hc, (((0,), (0,)), ((), ())),
                                   preferred_element_type=jnp.float32,
                                   precision=lax.Precision.HIGHEST)


def _fm_call(e2m, xvt, e1t, rmat, swmat):
    return pl.pallas_call(
        _fm_body,
        grid=(NT,),
        in_specs=[
            pl.BlockSpec((BT, F * D), lambda i: (i, 0)),
            pl.BlockSpec((F, BT), lambda i: (0, i)),
            pl.BlockSpec((F, BT), lambda i: (0, i)),
            pl.BlockSpec((F, F * D), lambda i: (0, 0)),
            pl.BlockSpec((F * D, D + H1), lambda i: (0, 0)),
        ],
        out_specs=[
            pl.BlockSpec((BT,), lambda i: (i,)),
            pl.BlockSpec((BT, H1), lambda i: (i, 0)),
            pl.BlockSpec((1, H1), lambda i: (0, 0)),
            pl.BlockSpec((H1, H1), lambda i: (0, 0)),
        ],
        out_shape=[
            jax.ShapeDtypeStruct((B,), jnp.float32),
            jax.ShapeDtypeStruct((B, H1), jnp.float32),
            jax.ShapeDtypeStruct((1, H1), jnp.float32),
            jax.ShapeDtypeStruct((H1, H1), jnp.float32),
        ],
    )(e2m, xvt, e1t, rmat, swmat)


def _fin_body(part0_ref, hc_ref, s1_ref, cc_ref, bias_ref, bl1_ref, g1_ref,
              b1_ref, wl2_ref, bl2_ref, g2_ref, b2_ref, out_ref):
    mc = s1_ref[...] * (1.0 / B)
    cc = cc_ref[...]
    eye = (lax.broadcasted_iota(jnp.int32, (H1, H1), 0)
           == lax.broadcasted_iota(jnp.int32, (H1, H1), 1)).astype(jnp.float32)
    diag = jnp.sum(cc * eye, axis=0, keepdims=True)
    v1 = diag * (1.0 / B) - mc * mc
    m1 = mc + bl1_ref[...]
    a = g1_ref[...] * lax.rsqrt(v1 + EPS)
    wl2 = wl2_ref[...]

    def rowvec_matT(x):
        return lax.dot_general(x, wl2, (((1,), (1,)), ((), ())),
                               preferred_element_type=jnp.float32,
                         precision=lax.Precision.HIGHEST)

    c = rowvec_matT(b1_ref[...] - m1 * a) + bl2_ref[...]
    m2 = rowvec_matT(m1 * a) + c
    outer_mc = lax.dot_general(mc, mc, (((0,), (0,)), ((), ())),
                               preferred_element_type=jnp.float32,
                         precision=lax.Precision.HIGHEST)
    cov = cc * (1.0 / B) - outer_mc
    outer_a = lax.dot_general(a, a, (((0,), (0,)), ((), ())),
                              preferred_element_type=jnp.float32,
                         precision=lax.Precision.HIGHEST)
    p = cov * outer_a
    q = lax.dot_general(wl2, p, (((1,), (0,)), ((), ())),
                        preferred_element_type=jnp.float32,
                         precision=lax.Precision.HIGHEST)
    v2 = jnp.sum(q * wl2, axis=1).reshape(1, H2)
    w2v = g2_ref[...] * lax.rsqrt(v2 + EPS)
    u = a * lax.dot_general(w2v, wl2, (((1,), (0,)), ((), ())),
                            preferred_element_type=jnp.float32,
                         precision=lax.Precision.HIGHEST)
    const = (bias_ref[0, 0]
             + jnp.sum((c - m2) * w2v)
             + jnp.sum(b2_ref[...])
             + jnp.sum(bl1_ref[...] * u))
    mv = lax.dot_general(hc_ref[...], u, (((1,), (1,)), ((), ())),
                         preferred_element_type=jnp.float32,
                         precision=lax.Precision.HIGHEST)
    out_ref[...] = part0_ref[...] + jnp.sum(mv, axis=1) + const


def _fin_call(part0, hc, s1, cc, bias, bl1, g1, b1, wl2, bl2, g2, b2):
    return pl.pallas_call(
        _fin_body,
        out_shape=jax.ShapeDtypeStruct((B,), jnp.float32),
    )(part0, hc, s1, cc, bias, bl1, g1, b1, wl2, bl2, g2, b2)


def kernel(Xi, Xv, W1, W2, bias, Wl1, bl1, g1, b1, Wl2, bl2, g2, b2):
    idxT = jnp.transpose(Xi[:, :, 0])
    fcol = jnp.arange(F, dtype=jnp.int32)[:, None]
    idxP = ((idxT // VC) * (VC * 8) + (idxT % VC) * 8 + fcol % 8)
    w2t2 = jnp.transpose(W2, (0, 2, 1)).reshape(F * D, V)
    w2l = _detile_call(w2t2).reshape(ROWS16, D)
    w1t = jnp.transpose(W1, (0, 2, 1))
    e2m = _sc_gather_e2()(idxP, w2l)
    e1T = _sc_gather_e1()(idxT, w1t)
    xvT = jnp.transpose(Xv)

    col = jnp.arange(F * D, dtype=jnp.int32)
    rmat = (col[None, :] // D == jnp.arange(F, dtype=jnp.int32)[:, None]
            ).astype(jnp.float32)
    smat = (col[:, None] % D == jnp.arange(D, dtype=jnp.int32)[None, :]
            ).astype(jnp.float32)
    swmat = jnp.concatenate([smat, Wl1.T], axis=1)

    part0, hc, s1, cc = _fm_call(e2m, xvT, e1T, rmat, swmat)
    out = _fin_call(part0, hc, s1, cc,
                    bias.reshape(1, 1), bl1.reshape(1, H1),
                    g1.reshape(1, H1), b1.reshape(1, H1), Wl2,
                    bl2.reshape(1, H2), g2.reshape(1, H2), b2.reshape(1, H2))
    return out
